# baseline (speedup 1.0000x reference)
import functools

import jax
import jax.numpy as jnp
from jax.experimental import pallas as pl
from jax.experimental.pallas import tpu as pltpu

_INTERPRET = False
_NB = 2
_NP = 432


def _vec2d(a):
    return a.reshape(1, -1)


def _bf16_pair(a):
    hi = a.astype(jnp.bfloat16)
    lo = (a - hi.astype(jnp.float32)).astype(jnp.bfloat16)
    return hi, lo


def _stack_w(a, axis=0):
    hi, lo = _bf16_pair(a)
    return jnp.concatenate([hi, lo, hi], axis=axis)


def _prep_rb(p):
    q = {
        'pre_g': _vec2d(p['pre_g']), 'pre_b': _vec2d(p['pre_b']),
        'lin1_W3': _stack_w(p['lin1_W'].T), 'lin1_b': _vec2d(p['lin1_b']),
        'n1_g': _vec2d(p['n1_g']), 'n1_b': _vec2d(p['n1_b']),
        'conv_W3': _stack_w(p['conv_W']), 'conv_b': _vec2d(p['conv_b']),
        'n2_g': _vec2d(p['n2_g']), 'n2_b': _vec2d(p['n2_b']),
        'lin2_W3': _stack_w(p['lin2_W'].T), 'lin2_b': _vec2d(p['lin2_b']),
    }
    if 'skip_W' in p:
        q['skip_W3'] = _stack_w(p['skip_W'].T)
        q['skip_b'] = _vec2d(p['skip_b'])
    return q


def _prep(params, nv):
    camw = params['cam_lin_W'].T
    camw = jnp.pad(camw, ((0, _NP - nv), (0, 0)))
    w0h, w0l = _bf16_pair(params['lin0_W'].T)
    return {
        'lin0_Wh': w0h, 'lin0_Wl': w0l,
        'lin0_Whs': (w0h.astype(jnp.float32) / 512.0).astype(jnp.bfloat16),
        'lin0_b': _vec2d(params['lin0_b']),
        'rb': [_prep_rb(p) for p in params['rb']],
        'shape_rb1': _prep_rb(params['shape_rb1']),
        'shape_rb2': _prep_rb(params['shape_rb2']),
        'shape_gn_g': _vec2d(params['shape_gn_g']),
        'shape_gn_b': _vec2d(params['shape_gn_b']),
        'shape_lin_W3': _stack_w(params['shape_lin_W'].T),
        'shape_lin_b': _vec2d(params['shape_lin_b']),
        'cam_gn_g': _vec2d(params['cam_gn_g']),
        'cam_gn_b': _vec2d(params['cam_gn_b']),
        'cam_glin_W3': _stack_w(params['cam_glin_W'].T),
        'cam_glin_b': _vec2d(params['cam_glin_b']),
        'cam_lin_W3': _stack_w(camw),
        'cam_lin_b': _vec2d(params['cam_lin_b']),
    }


def _dot(a, b, dims=((1,), (0,))):
    return jax.lax.dot_general(
        a, b, (dims, ((), ())), preferred_element_type=jnp.float32)


def _mm3(a, w3, contract_lhs=1):
    ah, al = _bf16_pair(a)
    a3 = jnp.concatenate([ah, ah, al], axis=contract_lhs)
    return _dot(a3, w3[...], (((contract_lhs,), (0,))))


def _adj_conv(adj3, t, ne):
    th, tl = _bf16_pair(t)
    outs = []
    for e in range(ne):
        lo, hi = e * _NP, (e + 1) * _NP
        t3 = jnp.concatenate([th[lo:hi], tl[lo:hi], th[lo:hi]], axis=0)
        outs.append(_dot(adj3, t3))
    return jnp.concatenate(outs, axis=0) if ne > 1 else outs[0]


def _relu(a):
    return jnp.maximum(a, 0.0)


def _group_norm(y, g, b, ne, nv, eps=1e-5, relu=True):
    c = y.shape[1]
    ng = c // 8
    halves = [y[e * _NP:(e + 1) * _NP] for e in range(ne)]
    s_rows, ss_rows = [], []
    for e, ye in enumerate(halves):
        pad_row = y[(e + 1) * _NP - 1:(e + 1) * _NP]
        s_rows.append(jnp.sum(ye, axis=0, keepdims=True) - pad_row)
        ss_rows.append(jnp.sum(ye * ye, axis=0, keepdims=True)
                       - pad_row * pad_row)
    stats = jnp.concatenate(s_rows + ss_rows, axis=0)
    ci = jax.lax.broadcasted_iota(jnp.int32, (c, ng), 0)
    gi = jax.lax.broadcasted_iota(jnp.int32, (c, ng), 1)
    sel = jnp.where(ci // 8 == gi, 1.0, 0.0).astype(jnp.bfloat16)
    sth, stl = _bf16_pair(stats)
    st = jnp.concatenate([sth, stl], axis=0)
    g4 = _dot(st, sel)
    cnt = 8.0 * nv
    gs = (g4[0:2 * ne] + g4[2 * ne:4 * ne]) / cnt
    mg = gs[0:ne]
    vg = gs[ne:2 * ne] - mg * mg
    ig = jax.lax.rsqrt(vg + eps)
    mi = jnp.concatenate([mg, ig], axis=0)
    mih, mil = _bf16_pair(mi)
    mi4 = jnp.concatenate([mih, mil], axis=0)
    bc4 = _dot(mi4, sel, (((1,), (1,))))
    bc = bc4[0:2 * ne] + bc4[2 * ne:4 * ne]
    outs = []
    for e, ye in enumerate(halves):
        scale = bc[ne + e:ne + e + 1] * g
        shift = b - bc[e:e + 1] * scale
        o = ye * scale + shift
        outs.append(_relu(o) if relu else o)
    return jnp.concatenate(outs, axis=0) if ne > 1 else outs[0]


def _resblock(y, rp, adj3, ne, nv):
    t = _group_norm(y, rp['pre_g'][...], rp['pre_b'][...], ne, nv)
    t = _mm3(t, rp['lin1_W3']) + rp['lin1_b'][...]
    t = _group_norm(t, rp['n1_g'][...], rp['n1_b'][...], ne, nv)
    t = _adj_conv(adj3, _mm3(t, rp['conv_W3']), ne) + rp['conv_b'][...]
    t = _group_norm(t, rp['n2_g'][...], rp['n2_b'][...], ne, nv)
    t = _mm3(t, rp['lin2_W3']) + rp['lin2_b'][...]
    if 'skip_W3' in rp:
        y = _mm3(y, rp['skip_W3']) + rp['skip_b'][...]
    return y + t


def _gcn_body(treedef, n_w, ne, nv, *refs):
    xh_ref, xl_ref, adj3_ref = refs[0], refs[1], refs[2]
    wrefs = refs[3:3 + n_w]
    shape_ref, cam_ref = refs[3 + n_w], refs[4 + n_w]
    p = jax.tree.unflatten(treedef, list(wrefs))
    adj3 = adj3_ref[...]

    xh = jnp.concatenate([xh_ref[e] for e in range(ne)], axis=1)
    xl = jnp.concatenate([xl_ref[e] for e in range(ne)],
                         axis=1).astype(jnp.bfloat16)
    d00 = ((0,), (0,))
    h = (_dot(xh, p['lin0_Wh'][...], d00) + _dot(xh, p['lin0_Wl'][...], d00)
         + _dot(xl, p['lin0_Whs'][...], d00) + p['lin0_b'][...])

    for rp in p['rb']:
        h = _resblock(h, rp, adj3, ne, nv)

    s = _resblock(h, p['shape_rb1'], adj3, ne, nv)
    s = _resblock(s, p['shape_rb2'], adj3, ne, nv)
    s = _group_norm(s, p['shape_gn_g'][...], p['shape_gn_b'][...], ne, nv)
    so = _mm3(s, p['shape_lin_W3']) + p['shape_lin_b'][...]

    c = _group_norm(h, p['cam_gn_g'][...], p['cam_gn_b'][...], ne, nv)
    c = _relu(_mm3(c, p['cam_glin_W3']) + p['cam_glin_b'][...])
    for e in range(ne):
        ce = c[e * _NP:(e + 1) * _NP]
        cam = _mm3(ce, p['cam_lin_W3'], contract_lhs=0) + p['cam_lin_b'][...]
        shape_ref[e] = so[e * _NP:e * _NP + nv]
        cam_ref[e] = cam


def kernel(x, params, adj):
    bsz, cin0, nv = x.shape
    nb = _NB if bsz % _NB == 0 else 1
    tp = _prep(params, nv)
    adjp = jnp.pad(adj, ((0, _NP - nv), (0, _NP - nv)))
    adjh, adjl = _bf16_pair(adjp)
    adj3 = jnp.concatenate([adjh, adjh, adjl], axis=1)
    xh32 = x.astype(jnp.bfloat16).astype(jnp.float32)
    xhp = jnp.pad(x.astype(jnp.bfloat16), ((0, 0), (0, 0), (0, _NP - nv)))
    xlp = jnp.pad(((x - xh32) * 512.0).astype(jnp.float8_e4m3fn),
                  ((0, 0), (0, 0), (0, _NP - nv)))
    leaves, treedef = jax.tree_util.tree_flatten(tp)
    n_w = len(leaves)

    in_specs = [
        pl.BlockSpec((nb, cin0, _NP), lambda b: (b, 0, 0)),
        pl.BlockSpec((nb, cin0, _NP), lambda b: (b, 0, 0)),
        pl.BlockSpec(adj3.shape, lambda b: (0, 0)),
    ]
    for leaf in leaves:
        in_specs.append(pl.BlockSpec(leaf.shape, lambda b: (0, 0)))

    out_shapes = [
        jax.ShapeDtypeStruct((bsz, nv, 3), jnp.float32),
        jax.ShapeDtypeStruct((bsz, 1, 3), jnp.float32),
    ]
    out_specs = [
        pl.BlockSpec((nb, nv, 3), lambda b: (b, 0, 0)),
        pl.BlockSpec((nb, 1, 3), lambda b: (b, 0, 0)),
    ]

    shape_k, cam_k = pl.pallas_call(
        functools.partial(_gcn_body, treedef, n_w, nb, nv),
        grid=(bsz // nb,),
        in_specs=in_specs,
        out_specs=out_specs,
        out_shape=out_shapes,
        compiler_params=pltpu.CompilerParams(
            dimension_semantics=("parallel",)),
        interpret=_INTERPRET,
    )(xhp, xlp, adj3, *leaves)

    return jnp.swapaxes(shape_k, 1, 2), cam_k.reshape(bsz, 3)


# speedup vs baseline: 1.7445x; 1.7445x over previous
import functools

import jax
import jax.numpy as jnp
from jax.experimental import pallas as pl
from jax.experimental.pallas import tpu as pltpu

_INTERPRET = False
_NB = 2
_NP = 432


def _vec2d(a):
    return a.reshape(1, -1)


def _bf16_pair(a):
    hi = a.astype(jnp.bfloat16)
    lo = (a - hi.astype(jnp.float32)).astype(jnp.bfloat16)
    return hi, lo


def _stack_w(a, axis=0):
    hi, lo = _bf16_pair(a)
    return jnp.concatenate([hi, lo, hi], axis=axis)


def _prep_rb(p):
    q = {
        'pre_g': _vec2d(p['pre_g']), 'pre_b': _vec2d(p['pre_b']),
        'lin1_W3': _stack_w(p['lin1_W'].T), 'lin1_b': _vec2d(p['lin1_b']),
        'n1_g': _vec2d(p['n1_g']), 'n1_b': _vec2d(p['n1_b']),
        'conv_W3': _stack_w(p['conv_W']), 'conv_b': _vec2d(p['conv_b']),
        'n2_g': _vec2d(p['n2_g']), 'n2_b': _vec2d(p['n2_b']),
        'lin2_W3': _stack_w(p['lin2_W'].T), 'lin2_b': _vec2d(p['lin2_b']),
    }
    if 'skip_W' in p:
        q['skip_W3'] = _stack_w(p['skip_W'].T)
        q['skip_b'] = _vec2d(p['skip_b'])
    return q


def _prep(params, nv):
    camw = params['cam_lin_W'].T
    camw = jnp.pad(camw, ((0, _NP - nv), (0, 0)))
    w0h, w0l = _bf16_pair(params['lin0_W'].T)
    return {
        'lin0_Wh': w0h, 'lin0_Wl': w0l,
        'lin0_b': _vec2d(params['lin0_b']),
        'rb': [_prep_rb(p) for p in params['rb']],
        'shape_rb1': _prep_rb(params['shape_rb1']),
        'shape_rb2': _prep_rb(params['shape_rb2']),
        'shape_gn_g': _vec2d(params['shape_gn_g']),
        'shape_gn_b': _vec2d(params['shape_gn_b']),
        'shape_lin_W3': _stack_w(params['shape_lin_W'].T),
        'shape_lin_b': _vec2d(params['shape_lin_b']),
        'cam_gn_g': _vec2d(params['cam_gn_g']),
        'cam_gn_b': _vec2d(params['cam_gn_b']),
        'cam_glin_W3': _stack_w(params['cam_glin_W'].T),
        'cam_glin_b': _vec2d(params['cam_glin_b']),
        'cam_lin_W3': _stack_w(camw),
        'cam_lin_b': _vec2d(params['cam_lin_b']),
    }


def _dot(a, b, dims=((1,), (0,))):
    return jax.lax.dot_general(
        a, b, (dims, ((), ())), preferred_element_type=jnp.float32)


def _mm3(a, w3, contract_lhs=1):
    ah, al = _bf16_pair(a)
    a3 = jnp.concatenate([ah, ah, al], axis=contract_lhs)
    return _dot(a3, w3[...], (((contract_lhs,), (0,))))


def _adj_conv(adj3, t, ne):
    th, tl = _bf16_pair(t)
    outs = []
    for e in range(ne):
        lo, hi = e * _NP, (e + 1) * _NP
        t3 = jnp.concatenate([th[lo:hi], tl[lo:hi], th[lo:hi]], axis=0)
        outs.append(_dot(adj3, t3))
    return jnp.concatenate(outs, axis=0) if ne > 1 else outs[0]


def _relu(a):
    return jnp.maximum(a, 0.0)


def _group_norm(y, g, b, ne, nv, eps=1e-5, relu=True):
    c = y.shape[1]
    ng = c // 8
    halves = [y[e * _NP:(e + 1) * _NP] for e in range(ne)]
    s_rows, ss_rows = [], []
    for e, ye in enumerate(halves):
        pad_row = y[(e + 1) * _NP - 1:(e + 1) * _NP]
        s_rows.append(jnp.sum(ye, axis=0, keepdims=True) - pad_row)
        ss_rows.append(jnp.sum(ye * ye, axis=0, keepdims=True)
                       - pad_row * pad_row)
    stats = jnp.concatenate(s_rows + ss_rows, axis=0)
    ci = jax.lax.broadcasted_iota(jnp.int32, (c, ng), 0)
    gi = jax.lax.broadcasted_iota(jnp.int32, (c, ng), 1)
    sel = jnp.where(ci // 8 == gi, 1.0, 0.0).astype(jnp.bfloat16)
    sth, stl = _bf16_pair(stats)
    st = jnp.concatenate([sth, stl], axis=0)
    g4 = _dot(st, sel)
    cnt = 8.0 * nv
    gs = (g4[0:2 * ne] + g4[2 * ne:4 * ne]) / cnt
    mg = gs[0:ne]
    vg = gs[ne:2 * ne] - mg * mg
    ig = jax.lax.rsqrt(vg + eps)
    mi = jnp.concatenate([mg, ig], axis=0)
    mih, mil = _bf16_pair(mi)
    mi4 = jnp.concatenate([mih, mil], axis=0)
    bc4 = _dot(mi4, sel, (((1,), (1,))))
    bc = bc4[0:2 * ne] + bc4[2 * ne:4 * ne]
    outs = []
    for e, ye in enumerate(halves):
        scale = bc[ne + e:ne + e + 1] * g
        shift = b - bc[e:e + 1] * scale
        o = ye * scale + shift
        outs.append(_relu(o) if relu else o)
    return jnp.concatenate(outs, axis=0) if ne > 1 else outs[0]


def _resblock(y, rp, adj3, ne, nv):
    t = _group_norm(y, rp['pre_g'][...], rp['pre_b'][...], ne, nv)
    t = _mm3(t, rp['lin1_W3']) + rp['lin1_b'][...]
    t = _group_norm(t, rp['n1_g'][...], rp['n1_b'][...], ne, nv)
    t = _adj_conv(adj3, _mm3(t, rp['conv_W3']), ne) + rp['conv_b'][...]
    t = _group_norm(t, rp['n2_g'][...], rp['n2_b'][...], ne, nv)
    t = _mm3(t, rp['lin2_W3']) + rp['lin2_b'][...]
    if 'skip_W3' in rp:
        y = _mm3(y, rp['skip_W3']) + rp['skip_b'][...]
    return y + t


def _gcn_body(treedef, n_w, ne, nv, *refs):
    xh_ref, adj3_ref = refs[0], refs[1]
    wrefs = refs[2:2 + n_w]
    shape_ref, cam_ref = refs[2 + n_w], refs[3 + n_w]
    p = jax.tree.unflatten(treedef, list(wrefs))
    adj3 = adj3_ref[...]

    zc = jnp.zeros((xh_ref.shape[1], _NP - nv), jnp.float32)
    pieces = []
    for e in range(ne):
        pieces.extend([xh_ref[e], zc])
    xcat = jnp.concatenate(pieces, axis=1)
    xh, xl = _bf16_pair(xcat)
    d00 = ((0,), (0,))
    w0h = p['lin0_Wh'][...]
    h = (_dot(xh, w0h, d00) + _dot(xh, p['lin0_Wl'][...], d00)
         + _dot(xl, w0h, d00) + p['lin0_b'][...])

    for rp in p['rb']:
        h = _resblock(h, rp, adj3, ne, nv)

    s = _resblock(h, p['shape_rb1'], adj3, ne, nv)
    s = _resblock(s, p['shape_rb2'], adj3, ne, nv)
    s = _group_norm(s, p['shape_gn_g'][...], p['shape_gn_b'][...], ne, nv)
    so = _mm3(s, p['shape_lin_W3']) + p['shape_lin_b'][...]

    c = _group_norm(h, p['cam_gn_g'][...], p['cam_gn_b'][...], ne, nv)
    c = _relu(_mm3(c, p['cam_glin_W3']) + p['cam_glin_b'][...])
    for e in range(ne):
        ce = c[e * _NP:(e + 1) * _NP]
        cam = _mm3(ce, p['cam_lin_W3'], contract_lhs=0) + p['cam_lin_b'][...]
        shape_ref[e] = so[e * _NP:e * _NP + nv]
        cam_ref[e] = cam


def kernel(x, params, adj):
    bsz, cin0, nv = x.shape
    nb = _NB if bsz % _NB == 0 else 1
    tp = _prep(params, nv)
    adjp = jnp.pad(adj, ((0, _NP - nv), (0, _NP - nv)))
    adjh, adjl = _bf16_pair(adjp)
    adj3 = jnp.concatenate([adjh, adjh, adjl], axis=1)
    leaves, treedef = jax.tree_util.tree_flatten(tp)
    n_w = len(leaves)

    in_specs = [
        pl.BlockSpec((nb, cin0, nv), lambda b: (b, 0, 0)),
        pl.BlockSpec(adj3.shape, lambda b: (0, 0)),
    ]
    for leaf in leaves:
        in_specs.append(pl.BlockSpec(leaf.shape, lambda b: (0, 0)))

    out_shapes = [
        jax.ShapeDtypeStruct((bsz, nv, 3), jnp.float32),
        jax.ShapeDtypeStruct((bsz, 1, 3), jnp.float32),
    ]
    out_specs = [
        pl.BlockSpec((nb, nv, 3), lambda b: (b, 0, 0)),
        pl.BlockSpec((nb, 1, 3), lambda b: (b, 0, 0)),
    ]

    shape_k, cam_k = pl.pallas_call(
        functools.partial(_gcn_body, treedef, n_w, nb, nv),
        grid=(bsz // nb,),
        in_specs=in_specs,
        out_specs=out_specs,
        out_shape=out_shapes,
        compiler_params=pltpu.CompilerParams(
            dimension_semantics=("parallel",)),
        interpret=_INTERPRET,
    )(x, adj3, *leaves)

    return jnp.swapaxes(shape_k, 1, 2), cam_k.reshape(bsz, 3)


# speedup vs baseline: 1.7482x; 1.0021x over previous
import functools

import jax
import jax.numpy as jnp
from jax.experimental import pallas as pl
from jax.experimental.pallas import tpu as pltpu

_NB = 2
_NP = 432


def _vec2d(a):
    return a.reshape(1, -1)


def _bf16_pair(a):
    hi = a.astype(jnp.bfloat16)
    lo = (a - hi.astype(jnp.float32)).astype(jnp.bfloat16)
    return hi, lo


def _stack_w(a, axis=0):
    hi, lo = _bf16_pair(a)
    return jnp.concatenate([hi, lo, hi], axis=axis)


def _prep_rb(p):
    q = {
        'pre_g': _vec2d(p['pre_g']), 'pre_b': _vec2d(p['pre_b']),
        'lin1_W3': _stack_w(p['lin1_W'].T), 'lin1_b': _vec2d(p['lin1_b']),
        'n1_g': _vec2d(p['n1_g']), 'n1_b': _vec2d(p['n1_b']),
        'conv_W3': _stack_w(p['conv_W']), 'conv_b': _vec2d(p['conv_b']),
        'n2_g': _vec2d(p['n2_g']), 'n2_b': _vec2d(p['n2_b']),
        'lin2_W3': _stack_w(p['lin2_W'].T), 'lin2_b': _vec2d(p['lin2_b']),
    }
    if 'skip_W' in p:
        q['skip_W3'] = _stack_w(p['skip_W'].T)
        q['skip_b'] = _vec2d(p['skip_b'])
    return q


def _prep(params, nv):
    camw = params['cam_lin_W'].T
    camw = jnp.pad(camw, ((0, _NP - nv), (0, 0)))
    w0h, w0l = _bf16_pair(params['lin0_W'].T)
    return {
        'lin0_Wh': w0h, 'lin0_Wl': w0l,
        'lin0_b': _vec2d(params['lin0_b']),
        'rb': [_prep_rb(p) for p in params['rb']],
        'shape_rb1': _prep_rb(params['shape_rb1']),
        'shape_rb2': _prep_rb(params['shape_rb2']),
        'shape_gn_g': _vec2d(params['shape_gn_g']),
        'shape_gn_b': _vec2d(params['shape_gn_b']),
        'shape_lin_W3': _stack_w(params['shape_lin_W'].T),
        'shape_lin_b': _vec2d(params['shape_lin_b']),
        'cam_gn_g': _vec2d(params['cam_gn_g']),
        'cam_gn_b': _vec2d(params['cam_gn_b']),
        'cam_glin_W3': _stack_w(params['cam_glin_W'].T),
        'cam_glin_b': _vec2d(params['cam_glin_b']),
        'cam_lin_W3': _stack_w(camw),
        'cam_lin_b': _vec2d(params['cam_lin_b']),
    }


def _dot(a, b, dims=((1,), (0,))):
    return jax.lax.dot_general(
        a, b, (dims, ((), ())), preferred_element_type=jnp.float32)


def _mm3(a, w3, contract_lhs=1):
    ah, al = _bf16_pair(a)
    a3 = jnp.concatenate([ah, ah, al], axis=contract_lhs)
    return _dot(a3, w3[...], (((contract_lhs,), (0,))))


def _adj_conv(adj3, t, ne):
    th, tl = _bf16_pair(t)
    outs = []
    for e in range(ne):
        lo, hi = e * _NP, (e + 1) * _NP
        t3 = jnp.concatenate([th[lo:hi], tl[lo:hi], th[lo:hi]], axis=0)
        outs.append(_dot(adj3, t3))
    return jnp.concatenate(outs, axis=0) if ne > 1 else outs[0]


def _relu(a):
    return jnp.maximum(a, 0.0)


def _group_norm(y, g, b, ne, nv, eps=1e-5, relu=True):
    c = y.shape[1]
    ng = c // 8
    halves = [y[e * _NP:(e + 1) * _NP] for e in range(ne)]
    s_rows, ss_rows = [], []
    for e, ye in enumerate(halves):
        pad_row = y[(e + 1) * _NP - 1:(e + 1) * _NP]
        s_rows.append(jnp.sum(ye, axis=0, keepdims=True) - pad_row)
        ss_rows.append(jnp.sum(ye * ye, axis=0, keepdims=True)
                       - pad_row * pad_row)
    stats = jnp.concatenate(s_rows + ss_rows, axis=0)
    ci = jax.lax.broadcasted_iota(jnp.int32, (c, ng), 0)
    gi = jax.lax.broadcasted_iota(jnp.int32, (c, ng), 1)
    sel = jnp.where(ci // 8 == gi, 1.0, 0.0).astype(jnp.bfloat16)
    sth, stl = _bf16_pair(stats)
    st = jnp.concatenate([sth, stl], axis=0)
    g4 = _dot(st, sel)
    cnt = 8.0 * nv
    gs = (g4[0:2 * ne] + g4[2 * ne:4 * ne]) / cnt
    mg = gs[0:ne]
    vg = gs[ne:2 * ne] - mg * mg
    ig = jax.lax.rsqrt(vg + eps)
    mi = jnp.concatenate([mg, ig], axis=0)
    mih, mil = _bf16_pair(mi)
    mi4 = jnp.concatenate([mih, mil], axis=0)
    bc4 = _dot(mi4, sel, (((1,), (1,))))
    bc = bc4[0:2 * ne] + bc4[2 * ne:4 * ne]
    outs = []
    for e, ye in enumerate(halves):
        scale = bc[ne + e:ne + e + 1] * g
        shift = b - bc[e:e + 1] * scale
        o = ye * scale + shift
        outs.append(_relu(o) if relu else o)
    return jnp.concatenate(outs, axis=0) if ne > 1 else outs[0]


def _resblock(y, rp, adj3, ne, nv):
    t = _group_norm(y, rp['pre_g'][...], rp['pre_b'][...], ne, nv)
    t = _mm3(t, rp['lin1_W3']) + rp['lin1_b'][...]
    t = _group_norm(t, rp['n1_g'][...], rp['n1_b'][...], ne, nv)
    t = _adj_conv(adj3, _mm3(t, rp['conv_W3']), ne) + rp['conv_b'][...]
    t = _group_norm(t, rp['n2_g'][...], rp['n2_b'][...], ne, nv)
    t = _mm3(t, rp['lin2_W3']) + rp['lin2_b'][...]
    if 'skip_W3' in rp:
        y = _mm3(y, rp['skip_W3']) + rp['skip_b'][...]
    return y + t


def _gcn_body(treedef, n_w, ne, nv, *refs):
    xh_ref, adj3_ref = refs[0], refs[1]
    wrefs = refs[2:2 + n_w]
    shape_ref, cam_ref = refs[2 + n_w], refs[3 + n_w]
    p = jax.tree.unflatten(treedef, list(wrefs))
    adj3 = adj3_ref[...]

    zc = jnp.zeros((xh_ref.shape[1], _NP - nv), jnp.float32)
    pieces = []
    for e in range(ne):
        pieces.extend([xh_ref[e], zc])
    xcat = jnp.concatenate(pieces, axis=1)
    xh, xl = _bf16_pair(xcat)
    d00 = ((0,), (0,))
    w0h = p['lin0_Wh'][...]
    h = (_dot(xh, w0h, d00) + _dot(xh, p['lin0_Wl'][...], d00)
         + _dot(xl, w0h, d00) + p['lin0_b'][...])

    for rp in p['rb']:
        h = _resblock(h, rp, adj3, ne, nv)

    s = _resblock(h, p['shape_rb1'], adj3, ne, nv)
    s = _resblock(s, p['shape_rb2'], adj3, ne, nv)
    s = _group_norm(s, p['shape_gn_g'][...], p['shape_gn_b'][...], ne, nv)
    so = _mm3(s, p['shape_lin_W3']) + p['shape_lin_b'][...]

    c = _group_norm(h, p['cam_gn_g'][...], p['cam_gn_b'][...], ne, nv)
    c = _relu(_mm3(c, p['cam_glin_W3']) + p['cam_glin_b'][...])
    for e in range(ne):
        ce = c[e * _NP:(e + 1) * _NP]
        cam = _mm3(ce, p['cam_lin_W3'], contract_lhs=0) + p['cam_lin_b'][...]
        shape_ref[e] = so[e * _NP:e * _NP + nv]
        cam_ref[e] = cam


def kernel(x, params, adj):
    bsz, cin0, nv = x.shape
    nb = _NB if bsz % _NB == 0 else 1
    tp = _prep(params, nv)
    adjp = jnp.pad(adj, ((0, _NP - nv), (0, _NP - nv)))
    adjh, adjl = _bf16_pair(adjp)
    adj3 = jnp.concatenate([adjh, adjh, adjl], axis=1)
    leaves, treedef = jax.tree_util.tree_flatten(tp)
    n_w = len(leaves)

    in_specs = [
        pl.BlockSpec((nb, cin0, nv), lambda b: (b, 0, 0)),
        pl.BlockSpec(adj3.shape, lambda b: (0, 0)),
    ]
    for leaf in leaves:
        in_specs.append(pl.BlockSpec(leaf.shape, lambda b: (0, 0)))

    out_shapes = [
        jax.ShapeDtypeStruct((bsz, nv, 3), jnp.float32),
        jax.ShapeDtypeStruct((bsz, 1, 3), jnp.float32),
    ]
    out_specs = [
        pl.BlockSpec((nb, nv, 3), lambda b: (b, 0, 0)),
        pl.BlockSpec((nb, 1, 3), lambda b: (b, 0, 0)),
    ]

    shape_k, cam_k = pl.pallas_call(
        functools.partial(_gcn_body, treedef, n_w, nb, nv),
        grid=(bsz // nb,),
        in_specs=in_specs,
        out_specs=out_specs,
        out_shape=out_shapes,
        compiler_params=pltpu.CompilerParams(
            dimension_semantics=("parallel",)),
    )(x, adj3, *leaves)

    return jnp.swapaxes(shape_k, 1, 2), cam_k.reshape(bsz, 3)
